# Initial kernel scaffold; baseline (speedup 1.0000x reference)
import jax
import jax.numpy as jnp
from jax.experimental import pallas as pl


def kernel(em_entity, W1, W2, W3, re_attention_weight, u, en_weight, re_weight, re_specific_attention, edge_index, edge_type):
    raise NotImplementedError("write your pallas kernel here")


# speedup vs baseline: 2.4573x; 2.4573x over previous
import functools

import jax
import jax.numpy as jnp
from jax import lax
from jax.experimental import pallas as pl
from jax.experimental.pallas import tpu as pltpu
from jax.experimental.pallas import tpu_sc as plsc

N = 10000
NREL = 500
E = 320000
D = 128
DH = D // 2

NC = 2
NS = 16
NW = NC * NS
EW = E // NS
CH = 80
NCHUNK = EW // CH
EW1 = E // NW
NCHUNK1 = EW1 // CH
RELP = 512
NPAD = 10240
RPT = NPAD // NS
ZROWS = 128
NLANE = 16
HALF = NPAD // 2
DUM = 256
ACCR = HALF + DUM
RPH = HALF // NS

_MESH = plsc.VectorSubcoreMesh(core_axis_name="c", subcore_axis_name="s")
_SC_PARAMS = pltpu.CompilerParams(needs_layout_passes=False,
                                 use_tc_tiling_on_sc=False)


def _fill_zero(zbuf_v):
    zv = jnp.zeros((NLANE,), jnp.float32)

    def zbody(k, carry):
        for t in range(DH // NLANE):
            zbuf_v[k, pl.ds(NLANE * t, NLANE)] = zv
        return carry

    lax.fori_loop(0, ZROWS, zbody, 0)


def _scale_scatter_loop(x_hbm, gidx_v, sidx_v, w_v, rows_v, acc_sh, sem,
                        nchunk):
    def chunk(k, carry):
        pltpu.async_copy(x_hbm.at[gidx_v.at[k]], rows_v, sem).wait()

        def gbody(g, c2):
            w16 = w_v[k, pl.ds(NLANE * g, NLANE)]
            for j in range(NLANE):
                wv = jnp.full((NLANE,), w16[j], jnp.float32)
                row = g * NLANE + j
                for t in range(DH // NLANE):
                    sl = pl.ds(NLANE * t, NLANE)
                    rows_v[row, sl] = rows_v[row, sl] * wv
            return c2

        lax.fori_loop(0, CH // NLANE, gbody, 0)
        pltpu.sync_copy(rows_v, acc_sh.at[sidx_v.at[k]], add=True)
        return carry

    lax.fori_loop(0, nchunk, chunk, 0)


def _two_phase(xs_c, gidx_v, sidx_v, sidx2_v, w_v, rows_v, zbuf_v, acc_sh,
               sem, out_hbm, c, s):
    dum_v = HALF + ((lax.iota(jnp.int32, NLANE) + s * NLANE) &
                    jnp.int32(DUM - 1))
    _fill_zero(zbuf_v)
    for h in range(2):
        lo = h * HALF

        def rbody(k, carry):
            for mm in range(CH // NLANE):
                sl = pl.ds(NLANE * mm, NLANE)
                loc = sidx_v[k, sl] - lo
                msk = (loc >= 0) & (loc < HALF)
                sidx2_v[k, sl] = jnp.where(msk, loc, dum_v)
            return carry

        lax.fori_loop(0, NCHUNK, rbody, 0)

        pltpu.sync_copy(zbuf_v, acc_sh.at[pl.ds(s * RPH, ZROWS)])
        pltpu.sync_copy(zbuf_v, acc_sh.at[pl.ds(s * RPH + ZROWS, ZROWS)])
        pltpu.sync_copy(zbuf_v.at[pl.ds(0, 64)],
                        acc_sh.at[pl.ds(s * RPH + 2 * ZROWS, 64)])
        plsc.subcore_barrier()

        _scale_scatter_loop(xs_c, gidx_v, sidx2_v, w_v, rows_v, acc_sh,
                            sem, NCHUNK)

        plsc.subcore_barrier()
        pltpu.sync_copy(acc_sh.at[pl.ds(s * RPH, RPH)],
                        out_hbm.at[c, pl.ds(lo + s * RPH, RPH)])
        plsc.subcore_barrier()


def _sc_layer_body(xs_hbm, gidx_hbm, sidx_hbm, et_hbm, rel_hbm, out_hbm,
                   gidx_v, sidx_v, sidx2_v, w_v, rel_v, rows_v, zbuf_v,
                   acc_sh, sem):
    c = lax.axis_index("c")
    s = lax.axis_index("s")

    pltpu.sync_copy(gidx_hbm.at[s], gidx_v)
    pltpu.sync_copy(sidx_hbm.at[s], sidx_v)
    pltpu.sync_copy(et_hbm.at[s], sidx2_v)
    pltpu.sync_copy(rel_hbm, rel_v)

    def wbody(k, carry):
        for m in range(CH // NLANE):
            sl = pl.ds(NLANE * m, NLANE)
            w_v[k, sl] = plsc.load_gather(rel_v, [sidx2_v[k, sl]])
        return carry

    lax.fori_loop(0, NCHUNK, wbody, 0)

    _two_phase(xs_hbm.at[c], gidx_v, sidx_v, sidx2_v, w_v, rows_v, zbuf_v,
               acc_sh, sem, out_hbm, c, s)


_sc_layer = functools.partial(
    pl.kernel,
    out_type=jax.ShapeDtypeStruct((NC, NPAD, DH), jnp.float32),
    mesh=_MESH,
    compiler_params=_SC_PARAMS,
    scratch_types=[
        pltpu.VMEM((NCHUNK, CH), jnp.int32),
        pltpu.VMEM((NCHUNK, CH), jnp.int32),
        pltpu.VMEM((NCHUNK, CH), jnp.int32),
        pltpu.VMEM((NCHUNK, CH), jnp.float32),
        pltpu.VMEM((RELP,), jnp.float32),
        pltpu.VMEM((CH, DH), jnp.float32),
        pltpu.VMEM((ZROWS, DH), jnp.float32),
        pltpu.VMEM_SHARED((ACCR, DH), jnp.float32),
        pltpu.SemaphoreType.DMA,
    ],
)(_sc_layer_body)


def _sc_att1_body(s1_hbm, s3_hbm, gidx_hbm, sidx_hbm, e_hbm, part_hbm,
                  gidx_v, sidx_v, s1_v, s3_v, e_v, prow_v):
    wid = lax.axis_index("c") * NS + lax.axis_index("s")

    pltpu.sync_copy(gidx_hbm.at[wid], gidx_v)
    pltpu.sync_copy(sidx_hbm.at[wid], sidx_v)
    pltpu.sync_copy(s1_hbm, s1_v)
    pltpu.sync_copy(s3_hbm, s3_v)

    neg = jnp.full((NLANE,), -1e30, jnp.float32)

    def chunk(k, m):
        for mm in range(CH // NLANE):
            sl = pl.ds(NLANE * mm, NLANE)
            a = plsc.load_gather(s1_v, [gidx_v[k, sl]])
            b = plsc.load_gather(s3_v, [sidx_v[k, sl]])
            e16 = a + b
            e16 = jnp.where(e16 >= 0.0, e16, e16 * 0.01)
            e_v[k, sl] = e16
            m = jnp.maximum(m, e16)
        return m

    m = lax.fori_loop(0, NCHUNK1, chunk, neg)
    mt = jnp.max(m)

    def chunk2(k, sv):
        for mm in range(CH // NLANE):
            sl = pl.ds(NLANE * mm, NLANE)
            sv = sv + jnp.exp(e_v[k, sl] - mt)
        return sv

    sv = lax.fori_loop(0, NCHUNK1, chunk2, jnp.zeros((NLANE,), jnp.float32))
    st = jnp.sum(sv)

    lanes = lax.iota(jnp.int32, NLANE)
    prow_v[...] = jnp.where(lanes == 0, mt, jnp.where(lanes == 1, st, 0.0))
    pltpu.sync_copy(prow_v, part_hbm.at[wid])
    pltpu.sync_copy(e_v, e_hbm.at[wid])


_sc_att1 = functools.partial(
    pl.kernel,
    out_type=(jax.ShapeDtypeStruct((NW, NCHUNK1, CH), jnp.float32),
              jax.ShapeDtypeStruct((NW, NLANE), jnp.float32)),
    mesh=_MESH,
    compiler_params=_SC_PARAMS,
    scratch_types=[
        pltpu.VMEM((NCHUNK1, CH), jnp.int32),
        pltpu.VMEM((NCHUNK1, CH), jnp.int32),
        pltpu.VMEM((N,), jnp.float32),
        pltpu.VMEM((N,), jnp.float32),
        pltpu.VMEM((NCHUNK1, CH), jnp.float32),
        pltpu.VMEM((NLANE,), jnp.float32),
    ],
)(_sc_att1_body)


def _sc_att2_body(xs_hbm, e_hbm, part_hbm, gidx_hbm, sidx_hbm, out_hbm,
                  gidx_v, sidx_v, sidx2_v, e_v, part_v, rows_v, zbuf_v,
                  acc_sh, sem):
    c = lax.axis_index("c")
    s = lax.axis_index("s")

    pltpu.sync_copy(gidx_hbm.at[s], gidx_v)
    pltpu.sync_copy(sidx_hbm.at[s], sidx_v)
    pltpu.sync_copy(e_hbm.at[s], e_v)
    pltpu.sync_copy(part_hbm, part_v)

    rows16 = lax.iota(jnp.int32, NLANE)
    col0 = jnp.zeros((NLANE,), jnp.int32)
    col1 = col0 + 1
    mlo = plsc.load_gather(part_v, [rows16, col0])
    mhi = plsc.load_gather(part_v, [rows16 + NLANE, col0])
    slo = plsc.load_gather(part_v, [rows16, col1])
    shi = plsc.load_gather(part_v, [rows16 + NLANE, col1])
    mg = jnp.max(jnp.maximum(mlo, mhi))
    sg = jnp.sum(jnp.exp(mlo - mg) * slo + jnp.exp(mhi - mg) * shi)
    rinv = jnp.ones((NLANE,), jnp.float32) / jnp.full((NLANE,), sg,
                                                      jnp.float32)

    def abody(k, carry):
        for mm in range(CH // NLANE):
            sl = pl.ds(NLANE * mm, NLANE)
            e_v[k, sl] = jnp.exp(e_v[k, sl] - mg) * rinv
        return carry

    lax.fori_loop(0, NCHUNK, abody, 0)

    _two_phase(xs_hbm.at[c], gidx_v, sidx_v, sidx2_v, e_v, rows_v, zbuf_v,
               acc_sh, sem, out_hbm, c, s)


_sc_att2 = functools.partial(
    pl.kernel,
    out_type=jax.ShapeDtypeStruct((NC, NPAD, DH), jnp.float32),
    mesh=_MESH,
    compiler_params=_SC_PARAMS,
    scratch_types=[
        pltpu.VMEM((NCHUNK, CH), jnp.int32),
        pltpu.VMEM((NCHUNK, CH), jnp.int32),
        pltpu.VMEM((NCHUNK, CH), jnp.int32),
        pltpu.VMEM((NCHUNK, CH), jnp.float32),
        pltpu.VMEM((NW, NLANE), jnp.float32),
        pltpu.VMEM((CH, DH), jnp.float32),
        pltpu.VMEM((ZROWS, DH), jnp.float32),
        pltpu.VMEM_SHARED((ACCR, DH), jnp.float32),
        pltpu.SemaphoreType.DMA,
    ],
)(_sc_att2_body)


BN = 512
GN = (N + BN - 1) // BN


def _tc_layer_kernel(p_ref, xs_ref, w_ref, o_ref):
    acc = jnp.concatenate([p_ref[0] + xs_ref[0], p_ref[1] + xs_ref[1]],
                          axis=1)
    y = jnp.dot(acc, w_ref[...], preferred_element_type=jnp.float32)
    y = jnp.maximum(y, 0.0)
    o_ref[0] = y[:, :DH]
    o_ref[1] = y[:, DH:]


_tc_layer = pl.pallas_call(
    _tc_layer_kernel,
    grid=(GN,),
    in_specs=[
        pl.BlockSpec((NC, BN, DH), lambda i: (0, i, 0)),
        pl.BlockSpec((NC, BN, DH), lambda i: (0, i, 0)),
        pl.BlockSpec((D, D), lambda i: (0, 0)),
    ],
    out_specs=pl.BlockSpec((NC, BN, DH), lambda i: (0, i, 0)),
    out_shape=jax.ShapeDtypeStruct((NC, N, DH), jnp.float32),
)


def _tc_proj_kernel(xs_ref, en_ref, u8_ref, u2_ref, rw_ref, rs_ref, s8_ref):
    x3 = jnp.concatenate([xs_ref[0], xs_ref[1]], axis=1)
    a = lax.dot_general(en_ref[...], u8_ref[...], (((0,), (0,)), ((), ())),
                        preferred_element_type=jnp.float32)
    s8 = jnp.dot(x3, a, preferred_element_type=jnp.float32)
    v = lax.dot_general(rs_ref[...], rw_ref[...], (((1,), (1,)), ((), ())),
                        preferred_element_type=jnp.float32)
    cscal = jnp.sum(u2_ref[...] * v)
    cols = lax.broadcasted_iota(jnp.int32, (BN, 8), 1)
    s8_ref[...] = s8 + jnp.where(cols == 0, cscal, 0.0)


_tc_proj = pl.pallas_call(
    _tc_proj_kernel,
    grid=(GN,),
    in_specs=[
        pl.BlockSpec((NC, BN, DH), lambda i: (0, i, 0)),
        pl.BlockSpec((D, D), lambda i: (0, 0)),
        pl.BlockSpec((D, 8), lambda i: (0, 0)),
        pl.BlockSpec((1, D), lambda i: (0, 0)),
        pl.BlockSpec((D, D), lambda i: (0, 0)),
        pl.BlockSpec((1, D), lambda i: (0, 0)),
    ],
    out_specs=pl.BlockSpec((BN, 8), lambda i: (i, 0)),
    out_shape=jax.ShapeDtypeStruct((N, 8), jnp.float32),
)


def kernel(em_entity, W1, W2, W3, re_attention_weight, u, en_weight,
           re_weight, re_specific_attention, edge_index, edge_type):
    src = edge_index[0].astype(jnp.int32)
    dst = edge_index[1].astype(jnp.int32)
    src16 = src.reshape(NS, NCHUNK, CH)
    dst16 = dst.reshape(NS, NCHUNK, CH)
    src32 = src.reshape(NW, NCHUNK1, CH)
    dst32 = dst.reshape(NW, NCHUNK1, CH)
    et16 = edge_type.astype(jnp.int32).reshape(NS, NCHUNK, CH)
    relp = jnp.pad(re_attention_weight, (0, RELP - NREL))

    xs0 = jnp.stack([em_entity[:, :DH], em_entity[:, DH:]], axis=0)
    ws = jnp.stack([W1, W2, W3])

    def _layer_step(xs, w):
        p = _sc_layer(xs, src16, dst16, et16, relp)
        return _tc_layer(p, xs, w), None

    x3s, _ = lax.scan(_layer_step, xs0, ws)

    u3d = u.reshape(3, D)
    u8 = jnp.pad(jnp.stack([u3d[0], u3d[2]], axis=1), ((0, 0), (0, 6)))
    s8 = _tc_proj(x3s, en_weight, u8, u3d[1].reshape(1, D),
                  re_weight, re_specific_attention.reshape(1, D))

    e, part = _sc_att1(s8[:, 0], s8[:, 1], src32, dst32)
    e16 = e.reshape(NS, NCHUNK, CH)
    pout = _sc_att2(x3s, e16, part, dst16, src16)
    return jnp.concatenate([pout[0, :N], pout[1, :N]], axis=1)
